# baseline (speedup 1.0000x reference)
import functools

import jax
import jax.numpy as jnp
from jax import lax
from jax.experimental import pallas as pl
from jax.experimental.pallas import tpu as pltpu
from jax.experimental.pallas import tpu_sc as plsc

_NC = 2
_NS = 16
_NW = _NC * _NS
_BD = 128


def _pad_rows(n):
    wb = (-(-n // _NS) + 7) // 8 * 8
    return wb, _NS * wb


def _sc_degree(col, zeros, n):
    e = col.shape[0]
    per = e // _NW
    nbw = per // _BD
    wb, n_pad = _pad_rows(n)
    mesh = plsc.VectorSubcoreMesh(core_axis_name="c", subcore_axis_name="s")

    @functools.partial(
        pl.kernel,
        mesh=mesh,
        out_type=jax.ShapeDtypeStruct((_NC * n_pad, 128), jnp.float32),
        scratch_types=[
            pltpu.VMEM((_BD,), jnp.int32),
            pltpu.VMEM((_BD, 128), jnp.float32),
            pltpu.VMEM_SHARED((n_pad, 128), jnp.float32),
        ],
    )
    def deg_kernel(col_hbm, zeros_hbm, out_hbm, cidx, ones_v, acc):
        c = lax.axis_index("c")
        s = lax.axis_index("s")
        wid = s * _NC + c
        off = s * wb

        def fill_ones(i, carry):
            ones_v[i // 8, pl.ds((i % 8) * 16, 16)] = jnp.ones((16,), jnp.float32)
            return carry

        lax.fori_loop(0, _BD * 8, fill_ones, 0)
        pltpu.sync_copy(zeros_hbm.at[pl.ds(0, wb)], acc.at[pl.ds(off, wb)])
        plsc.subcore_barrier()

        def step(j, carry):
            base = wid * per + j * _BD
            pltpu.sync_copy(col_hbm.at[pl.ds(base, _BD)], cidx)
            pltpu.sync_copy(ones_v, acc.at[cidx], add=True)
            return carry

        lax.fori_loop(0, nbw, step, 0)
        plsc.subcore_barrier()
        pltpu.sync_copy(acc.at[pl.ds(off, wb)],
                        out_hbm.at[pl.ds(c * n_pad + off, wb)])

    return deg_kernel(col, zeros).reshape(_NC, n_pad, 128)


def _sc_scatter(y, row, col, zeros, n):
    e = row.shape[0]
    d = y.shape[1]
    per = e // _NW
    nbw = per // _BD
    wb, n_pad = _pad_rows(n)
    mesh = plsc.VectorSubcoreMesh(core_axis_name="c", subcore_axis_name="s")

    @functools.partial(
        pl.kernel,
        mesh=mesh,
        out_type=jax.ShapeDtypeStruct((_NC * n_pad, d), jnp.float32),
        scratch_types=[
            pltpu.VMEM((_BD,), jnp.int32),
            pltpu.VMEM((_BD,), jnp.int32),
            pltpu.VMEM((_BD, d), jnp.float32),
            pltpu.VMEM_SHARED((n_pad, d), jnp.float32),
            pltpu.SemaphoreType.DMA,
        ],
    )
    def scat_kernel(y_hbm, row_hbm, col_hbm, zeros_hbm, out_hbm,
                    ridx, cidx, rows, acc, sem):
        c = lax.axis_index("c")
        s = lax.axis_index("s")
        wid = s * _NC + c
        off = s * wb

        pltpu.sync_copy(zeros_hbm.at[pl.ds(0, wb)], acc.at[pl.ds(off, wb)])
        plsc.subcore_barrier()

        def step(j, carry):
            base = wid * per + j * _BD
            pltpu.sync_copy(row_hbm.at[pl.ds(base, _BD)], ridx)
            pltpu.sync_copy(col_hbm.at[pl.ds(base, _BD)], cidx)
            pltpu.async_copy(y_hbm.at[ridx], rows, sem).wait()
            pltpu.sync_copy(rows, acc.at[cidx], add=True)
            return carry

        lax.fori_loop(0, nbw, step, 0)
        plsc.subcore_barrier()
        pltpu.sync_copy(acc.at[pl.ds(off, wb)],
                        out_hbm.at[pl.ds(c * n_pad + off, wb)])

    return scat_kernel(y, row, col, zeros).reshape(_NC, n_pad, d)


def _dinv_block(deg_ref):
    deg = deg_ref[0][:, :1] + deg_ref[1][:, :1] + 2.0
    return jnp.where(deg > 0, lax.rsqrt(deg), 0.0)


def _tc_first(x, w, degp):
    n, din = x.shape
    dh = w.shape[1]
    bm = 1000

    def body(x_ref, w_ref, deg_ref, y_ref):
        dinv = _dinv_block(deg_ref)
        y_ref[...] = dinv * jnp.dot(x_ref[...], w_ref[...],
                                    preferred_element_type=jnp.float32)

    return pl.pallas_call(
        body,
        grid=(n // bm,),
        in_specs=[
            pl.BlockSpec((bm, din), lambda i: (i, 0)),
            pl.BlockSpec((din, dh), lambda i: (0, 0)),
            pl.BlockSpec((2, bm, 128), lambda i: (0, i, 0)),
        ],
        out_specs=pl.BlockSpec((bm, dh), lambda i: (i, 0)),
        out_shape=jax.ShapeDtypeStruct((n, dh), jnp.float32),
    )(x, w, degp)


def _tc_mid(sp, y, degp, b, w):
    n, dh = y.shape
    bm = 1000

    def body(s_ref, y_ref, deg_ref, b_ref, w_ref, o_ref):
        dinv = _dinv_block(deg_ref)
        h = jnp.maximum(
            dinv * (s_ref[0] + s_ref[1] + 2.0 * y_ref[...]) + b_ref[...], 0.0)
        o_ref[...] = dinv * jnp.dot(h, w_ref[...],
                                    preferred_element_type=jnp.float32)

    return pl.pallas_call(
        body,
        grid=(n // bm,),
        in_specs=[
            pl.BlockSpec((2, bm, dh), lambda i: (0, i, 0)),
            pl.BlockSpec((bm, dh), lambda i: (i, 0)),
            pl.BlockSpec((2, bm, 128), lambda i: (0, i, 0)),
            pl.BlockSpec((1, dh), lambda i: (0, 0)),
            pl.BlockSpec((dh, dh), lambda i: (0, 0)),
        ],
        out_specs=pl.BlockSpec((bm, dh), lambda i: (i, 0)),
        out_shape=jax.ShapeDtypeStruct((n, dh), jnp.float32),
    )(sp, y, degp, b, w)


def _tc_final(sp, y, degp, b, lin_w, lin_b):
    n, dh = y.shape
    bm = 1000

    def body(s_ref, y_ref, deg_ref, b_ref, lw_ref, lb_ref, o_ref):
        dinv = _dinv_block(deg_ref)
        h = jnp.maximum(
            dinv * (s_ref[0] + s_ref[1] + 2.0 * y_ref[...]) + b_ref[...], 0.0)
        z = jnp.dot(h, lw_ref[...], preferred_element_type=jnp.float32)
        o_ref[...] = jax.nn.sigmoid(z + lb_ref[0, 0])

    return pl.pallas_call(
        body,
        grid=(n // bm,),
        in_specs=[
            pl.BlockSpec((2, bm, dh), lambda i: (0, i, 0)),
            pl.BlockSpec((bm, dh), lambda i: (i, 0)),
            pl.BlockSpec((2, bm, 128), lambda i: (0, i, 0)),
            pl.BlockSpec((1, dh), lambda i: (0, 0)),
            pl.BlockSpec((dh, 1), lambda i: (0, 0)),
            pl.BlockSpec((1, 1), lambda i: (0, 0)),
        ],
        out_specs=pl.BlockSpec((bm, 1), lambda i: (i, 0)),
        out_shape=jax.ShapeDtypeStruct((n, 1), jnp.float32),
    )(sp, y, degp, b, lin_w, lin_b)


def kernel(x, edge_index, W1, b1, W2, b2, W3, b3, lin_W, lin_b):
    n = x.shape[0]
    row = edge_index[0].astype(jnp.int32)
    col = edge_index[1].astype(jnp.int32)
    wb, n_pad = _pad_rows(n)
    zeros = jnp.zeros((wb, 128), jnp.float32)

    e = row.shape[0]
    chunk = _NW * _BD
    e_pad = -(-e // chunk) * chunk
    if e_pad != e:
        row = jnp.concatenate(
            [row, jnp.zeros((e_pad - e,), jnp.int32)])
        col = jnp.concatenate(
            [col, jnp.full((e_pad - e,), n_pad - 1, jnp.int32)])

    degp = _sc_degree(col, zeros, n)
    y1 = _tc_first(x, W1, degp)
    s1 = _sc_scatter(y1, row, col, zeros, n)
    y2 = _tc_mid(s1, y1, degp, b1.reshape(1, -1), W2)
    s2 = _sc_scatter(y2, row, col, zeros, n)
    y3 = _tc_mid(s2, y2, degp, b2.reshape(1, -1), W3)
    s3 = _sc_scatter(y3, row, col, zeros, n)
    return _tc_final(s3, y3, degp, b3.reshape(1, -1), lin_W,
                     lin_b.reshape(1, 1))


# speedup vs baseline: 1.4162x; 1.4162x over previous
import functools

import jax
import jax.numpy as jnp
from jax import lax
from jax.experimental import pallas as pl
from jax.experimental.pallas import tpu as pltpu
from jax.experimental.pallas import tpu_sc as plsc

_NC = 2
_NS = 16
_NW = _NC * _NS
_BD = 40


def _pad_rows(n):
    wb = (-(-n // _NS) + 7) // 8 * 8
    return wb, _NS * wb


def _sc_degree(col, zeros, n, e_pad):
    per = e_pad // _NW
    nbw = per // _BD
    wb, n_pad = _pad_rows(n)
    mesh = plsc.VectorSubcoreMesh(core_axis_name="c", subcore_axis_name="s")

    @functools.partial(
        pl.kernel,
        mesh=mesh,
        out_type=jax.ShapeDtypeStruct((_NC * n_pad, 128), jnp.float32),
        scratch_types=[
            pltpu.VMEM((_BD,), jnp.int32),
            pltpu.VMEM((_BD, 128), jnp.float32),
            pltpu.VMEM_SHARED((n_pad, 128), jnp.float32),
        ],
    )
    def deg_kernel(col_hbm, zeros_hbm, out_hbm, cidx, ones_v, acc):
        c = lax.axis_index("c")
        s = lax.axis_index("s")
        wid = s * _NC + c
        off = s * wb

        def fill_ones(i, carry):
            ones_v[i // 8, pl.ds((i % 8) * 16, 16)] = jnp.ones((16,), jnp.float32)
            return carry

        lax.fori_loop(0, _BD * 8, fill_ones, 0)
        pltpu.sync_copy(zeros_hbm.at[pl.ds(0, wb)], acc.at[pl.ds(off, wb)])
        plsc.subcore_barrier()

        def step(j, carry):
            base = wid * per + j * _BD
            pltpu.sync_copy(col_hbm.at[pl.ds(base, _BD)], cidx)
            pltpu.sync_copy(ones_v, acc.at[cidx], add=True)
            return carry

        lax.fori_loop(0, nbw, step, 0)
        plsc.subcore_barrier()
        pltpu.sync_copy(acc.at[pl.ds(off, wb)],
                        out_hbm.at[pl.ds(c * n_pad + off, wb)])

    return deg_kernel(col, zeros).reshape(_NC, n_pad, 128)


def _sc_scatter(y, row, col, zeros, n, e_pad):
    d = y.shape[1]
    per = e_pad // _NW
    nbw = per // _BD
    wb, n_pad = _pad_rows(n)
    mesh = plsc.VectorSubcoreMesh(core_axis_name="c", subcore_axis_name="s")

    @functools.partial(
        pl.kernel,
        mesh=mesh,
        out_type=jax.ShapeDtypeStruct((_NC * n_pad, d), jnp.float32),
        scratch_types=[
            pltpu.VMEM((_BD,), jnp.int32),
            pltpu.VMEM((_BD,), jnp.int32),
            pltpu.VMEM((_BD,), jnp.int32),
            pltpu.VMEM((_BD,), jnp.int32),
            pltpu.VMEM((_BD, d), jnp.float32),
            pltpu.VMEM((_BD, d), jnp.float32),
            pltpu.VMEM_SHARED((n_pad, d), jnp.float32),
            pltpu.SemaphoreType.DMA,
            pltpu.SemaphoreType.DMA,
        ],
    )
    def scat_kernel(y_hbm, row_hbm, col_hbm, zeros_hbm, out_hbm,
                    ridx_a, cidx_a, ridx_b, cidx_b, rows_a, rows_b,
                    acc, sem_a, sem_b):
        c = lax.axis_index("c")
        s = lax.axis_index("s")
        wid = s * _NC + c
        off = s * wb
        base0 = wid * per

        pltpu.sync_copy(zeros_hbm.at[pl.ds(0, wb)], acc.at[pl.ds(off, wb)])
        plsc.subcore_barrier()

        pltpu.sync_copy(row_hbm.at[pl.ds(base0, _BD)], ridx_a)
        pltpu.sync_copy(col_hbm.at[pl.ds(base0, _BD)], cidx_a)
        pltpu.async_copy(y_hbm.at[ridx_a], rows_a, sem_a)

        def step(i, carry):
            j = 2 * i
            pltpu.sync_copy(row_hbm.at[pl.ds(base0 + (j + 1) * _BD, _BD)], ridx_b)
            pltpu.sync_copy(col_hbm.at[pl.ds(base0 + (j + 1) * _BD, _BD)], cidx_b)
            pltpu.async_copy(y_hbm.at[ridx_b], rows_b, sem_b)
            pltpu.make_async_copy(y_hbm.at[ridx_a], rows_a, sem_a).wait()
            pltpu.sync_copy(rows_a, acc.at[cidx_a], add=True)
            pltpu.sync_copy(row_hbm.at[pl.ds(base0 + (j + 2) * _BD, _BD)], ridx_a)
            pltpu.sync_copy(col_hbm.at[pl.ds(base0 + (j + 2) * _BD, _BD)], cidx_a)
            pltpu.async_copy(y_hbm.at[ridx_a], rows_a, sem_a)
            pltpu.make_async_copy(y_hbm.at[ridx_b], rows_b, sem_b).wait()
            pltpu.sync_copy(rows_b, acc.at[cidx_b], add=True)
            return carry

        lax.fori_loop(0, nbw // 2, step, 0)
        pltpu.make_async_copy(y_hbm.at[ridx_a], rows_a, sem_a).wait()
        plsc.subcore_barrier()
        pltpu.sync_copy(acc.at[pl.ds(off, wb)],
                        out_hbm.at[pl.ds(c * n_pad + off, wb)])

    return scat_kernel(y, row, col, zeros).reshape(_NC, n_pad, d)


def _dinv_block(deg_ref):
    deg = deg_ref[0][:, :1] + deg_ref[1][:, :1] + 2.0
    return jnp.where(deg > 0, lax.rsqrt(deg), 0.0)


def _tc_first(x, w, degp):
    n, din = x.shape
    dh = w.shape[1]
    bm = 1000

    def body(x_ref, w_ref, deg_ref, y_ref):
        dinv = _dinv_block(deg_ref)
        y_ref[...] = dinv * jnp.dot(x_ref[...], w_ref[...],
                                    preferred_element_type=jnp.float32)

    return pl.pallas_call(
        body,
        grid=(n // bm,),
        in_specs=[
            pl.BlockSpec((bm, din), lambda i: (i, 0)),
            pl.BlockSpec((din, dh), lambda i: (0, 0)),
            pl.BlockSpec((2, bm, 128), lambda i: (0, i, 0)),
        ],
        out_specs=pl.BlockSpec((bm, dh), lambda i: (i, 0)),
        out_shape=jax.ShapeDtypeStruct((n, dh), jnp.float32),
    )(x, w, degp)


def _tc_mid(sp, y, degp, b, w):
    n, dh = y.shape
    bm = 1000

    def body(s_ref, y_ref, deg_ref, b_ref, w_ref, o_ref):
        dinv = _dinv_block(deg_ref)
        h = jnp.maximum(
            dinv * (s_ref[0] + s_ref[1] + 2.0 * y_ref[...]) + b_ref[...], 0.0)
        o_ref[...] = dinv * jnp.dot(h, w_ref[...],
                                    preferred_element_type=jnp.float32)

    return pl.pallas_call(
        body,
        grid=(n // bm,),
        in_specs=[
            pl.BlockSpec((2, bm, dh), lambda i: (0, i, 0)),
            pl.BlockSpec((bm, dh), lambda i: (i, 0)),
            pl.BlockSpec((2, bm, 128), lambda i: (0, i, 0)),
            pl.BlockSpec((1, dh), lambda i: (0, 0)),
            pl.BlockSpec((dh, dh), lambda i: (0, 0)),
        ],
        out_specs=pl.BlockSpec((bm, dh), lambda i: (i, 0)),
        out_shape=jax.ShapeDtypeStruct((n, dh), jnp.float32),
    )(sp, y, degp, b, w)


def _tc_final(sp, y, degp, b, lin_w, lin_b):
    n, dh = y.shape
    bm = 1000

    def body(s_ref, y_ref, deg_ref, b_ref, lw_ref, lb_ref, o_ref):
        dinv = _dinv_block(deg_ref)
        h = jnp.maximum(
            dinv * (s_ref[0] + s_ref[1] + 2.0 * y_ref[...]) + b_ref[...], 0.0)
        z = jnp.dot(h, lw_ref[...], preferred_element_type=jnp.float32)
        o_ref[...] = jax.nn.sigmoid(z + lb_ref[0, 0])

    return pl.pallas_call(
        body,
        grid=(n // bm,),
        in_specs=[
            pl.BlockSpec((2, bm, dh), lambda i: (0, i, 0)),
            pl.BlockSpec((bm, dh), lambda i: (i, 0)),
            pl.BlockSpec((2, bm, 128), lambda i: (0, i, 0)),
            pl.BlockSpec((1, dh), lambda i: (0, 0)),
            pl.BlockSpec((dh, 1), lambda i: (0, 0)),
            pl.BlockSpec((1, 1), lambda i: (0, 0)),
        ],
        out_specs=pl.BlockSpec((bm, 1), lambda i: (i, 0)),
        out_shape=jax.ShapeDtypeStruct((n, 1), jnp.float32),
    )(sp, y, degp, b, lin_w, lin_b)


def kernel(x, edge_index, W1, b1, W2, b2, W3, b3, lin_W, lin_b):
    n = x.shape[0]
    row = edge_index[0].astype(jnp.int32)
    col = edge_index[1].astype(jnp.int32)
    wb, n_pad = _pad_rows(n)
    zeros = jnp.zeros((wb, 128), jnp.float32)

    e = row.shape[0]
    chunk = _NW * 2 * _BD
    e_pad = -(-e // chunk) * chunk
    npad_e = e_pad - e + _BD
    row = jnp.concatenate([row, jnp.zeros((npad_e,), jnp.int32)])
    col = jnp.concatenate([col, jnp.full((npad_e,), n_pad - 1, jnp.int32)])

    degp = _sc_degree(col, zeros, n, e_pad)
    y1 = _tc_first(x, W1, degp)
    s1 = _sc_scatter(y1, row, col, zeros, n, e_pad)
    y2 = _tc_mid(s1, y1, degp, b1.reshape(1, -1), W2)
    s2 = _sc_scatter(y2, row, col, zeros, n, e_pad)
    y3 = _tc_mid(s2, y2, degp, b2.reshape(1, -1), W3)
    s3 = _sc_scatter(y3, row, col, zeros, n, e_pad)
    return _tc_final(s3, y3, degp, b3.reshape(1, -1), lin_W,
                     lin_b.reshape(1, 1))
